# baseline (speedup 1.0000x reference)
import jax
import jax.numpy as jnp
from jax import lax
from jax.experimental import pallas as pl
from jax.experimental.pallas import tpu as pltpu
from jax.experimental.pallas import tpu_sc as plsc

BATCH = 16384
EMBED_DIM = 128
N_TOTAL = float(BATCH * EMBED_DIM)

NC = 2
NS = 16
NW = NC * NS
LANES = 16
NCOLG = EMBED_DIM // LANES
SC_ROWS = 4096
TILE_ROWS = SC_ROWS // NW
CHUNK_ROWS = TILE_ROWS // 2
NCHUNK = TILE_ROWS // CHUNK_ROWS

TC_BLOCK_ROWS = 4096
TC_GRID = (BATCH - SC_ROWS) // TC_BLOCK_ROWS
TC_BLOCK_OFF = SC_ROWS // TC_BLOCK_ROWS


def _sc_tile_body(a_hbm, b_hbm, out_hbm, abuf, bbuf, stbuf, *sems):
    wid = lax.axis_index("s") * NC + lax.axis_index("c")
    base = wid * TILE_ROWS

    descs = [None, None]

    def issue(c, slot):
        off = base + c * CHUNK_ROWS
        da = pltpu.async_copy(a_hbm.at[pl.ds(off, CHUNK_ROWS), :],
                              abuf.at[slot], sems[2 * slot])
        db = pltpu.async_copy(b_hbm.at[pl.ds(off, CHUNK_ROWS), :],
                              bbuf.at[slot], sems[2 * slot + 1])
        descs[slot] = (da, db)

    zero = jnp.zeros((LANES,), jnp.float32)
    acc_e = (zero,) * NCOLG
    acc_m = (zero,) * NCOLG

    issue(0, 0)
    for c in range(NCHUNK):
        slot = c % 2
        if c + 1 < NCHUNK:
            issue(c + 1, (c + 1) % 2)
        da, db = descs[slot]
        da.wait()
        db.wait()

        def body(r, carry, _slot=slot):
            es, ms = carry
            new_es, new_ms = [], []
            for u in range(NCOLG):
                av = abuf[_slot, r, pl.ds(u * LANES, LANES)]
                bv = bbuf[_slot, r, pl.ds(u * LANES, LANES)]
                m = av * bv
                new_es.append(es[u] + jnp.exp(m))
                new_ms.append(ms[u] + m)
            return tuple(new_es), tuple(new_ms)

        acc_e, acc_m = lax.fori_loop(0, CHUNK_ROWS, body, (acc_e, acc_m))

    sum_e = zero
    sum_m = zero
    for u in range(NCOLG):
        sum_e = sum_e + acc_e[u]
        sum_m = sum_m + acc_m[u]

    stbuf[0, :] = sum_e
    stbuf[1, :] = sum_m
    pltpu.sync_copy(stbuf, out_hbm.at[wid])


def _tc_kernel(cw_ref, xw_ref, out_ref, acc_ref):
    i = pl.program_id(0)

    @pl.when(i == 0)
    def _init():
        acc_ref[0] = 0.0
        acc_ref[1] = 0.0

    mul = cw_ref[...] * xw_ref[...]
    acc_ref[0] += jnp.sum(jnp.exp(mul))
    acc_ref[1] += jnp.sum(mul)

    @pl.when(i == TC_GRID - 1)
    def _fini():
        out_ref[0] = acc_ref[0]
        out_ref[1] = acc_ref[1]


@jax.jit
def kernel(center_word, context_word, center_emb, context_emb):
    del center_emb, context_emb

    sc_call = pl.kernel(
        _sc_tile_body,
        out_type=jax.ShapeDtypeStruct((NW, 2, LANES), jnp.float32),
        mesh=plsc.VectorSubcoreMesh(core_axis_name="c", subcore_axis_name="s"),
        scratch_types=[
            pltpu.VMEM((2, CHUNK_ROWS, EMBED_DIM), jnp.float32),
            pltpu.VMEM((2, CHUNK_ROWS, EMBED_DIM), jnp.float32),
            pltpu.VMEM((2, LANES), jnp.float32),
            pltpu.SemaphoreType.DMA,
            pltpu.SemaphoreType.DMA,
            pltpu.SemaphoreType.DMA,
            pltpu.SemaphoreType.DMA,
        ],
    )
    sc_partials = sc_call(center_word, context_word)

    tc_partials = pl.pallas_call(
        _tc_kernel,
        grid=(TC_GRID,),
        in_specs=[
            pl.BlockSpec((TC_BLOCK_ROWS, EMBED_DIM),
                         lambda i: (i + TC_BLOCK_OFF, 0)),
            pl.BlockSpec((TC_BLOCK_ROWS, EMBED_DIM),
                         lambda i: (i + TC_BLOCK_OFF, 0)),
        ],
        out_specs=pl.BlockSpec(memory_space=pltpu.SMEM),
        out_shape=jax.ShapeDtypeStruct((2,), jnp.float32),
        scratch_shapes=[pltpu.SMEM((2,), jnp.float32)],
    )(center_word, context_word)

    sum_exp = tc_partials[0] + jnp.sum(sc_partials[:, 0, :])
    sum_mul = tc_partials[1] + jnp.sum(sc_partials[:, 1, :])
    return N_TOTAL * jnp.log(sum_exp) - sum_mul


# speedup vs baseline: 3.7500x; 3.7500x over previous
import jax
import jax.numpy as jnp
from jax.experimental import pallas as pl
from jax.experimental.pallas import tpu as pltpu

BATCH = 16384
EMBED_DIM = 128
N_TOTAL = float(BATCH * EMBED_DIM)
BLOCK_ROWS = 8192
GRID = BATCH // BLOCK_ROWS


def _nll_kernel(cw_ref, xw_ref, out_ref, acc_ref):
    i = pl.program_id(0)

    @pl.when(i == 0)
    def _init():
        acc_ref[0] = 0.0
        acc_ref[1] = 0.0

    mul = cw_ref[...] * xw_ref[...]
    acc_ref[0] += jnp.sum(jnp.exp(mul))
    acc_ref[1] += jnp.sum(mul)

    @pl.when(i == GRID - 1)
    def _fini():
        out_ref[0] = N_TOTAL * jnp.log(acc_ref[0]) - acc_ref[1]


@jax.jit
def kernel(center_word, context_word, center_emb, context_emb):
    del center_emb, context_emb
    out = pl.pallas_call(
        _nll_kernel,
        grid=(GRID,),
        in_specs=[
            pl.BlockSpec((BLOCK_ROWS, EMBED_DIM), lambda i: (i, 0)),
            pl.BlockSpec((BLOCK_ROWS, EMBED_DIM), lambda i: (i, 0)),
        ],
        out_specs=pl.BlockSpec(memory_space=pltpu.SMEM),
        out_shape=jax.ShapeDtypeStruct((1,), jnp.float32),
        scratch_shapes=[pltpu.SMEM((2,), jnp.float32)],
    )(center_word, context_word)
    return out[0]
